# Initial kernel scaffold; baseline (speedup 1.0000x reference)
import jax
import jax.numpy as jnp
from jax.experimental import pallas as pl


def kernel(x, time_table, channel_table):
    raise NotImplementedError("write your pallas kernel here")


# speedup vs baseline: 1.7669x; 1.7669x over previous
import jax
import jax.numpy as jnp
from jax.experimental import pallas as pl


def _body(x_ref, tt_ref, ct_ref, o_ref):
    q, d = tt_ref.shape
    c = ct_ref.shape[0]
    xv = x_ref[0].reshape(q, c, d)
    tt = tt_ref[...]
    ct = ct_ref[...]
    o_ref[0] = (xv + tt[:, None, :] + ct[None, :, :]).reshape(q * c, d)


def kernel(x, time_table, channel_table):
    b, c, T, D = x.shape
    assert T % c == 0
    q = T // c
    grid = (b, c)
    out = pl.pallas_call(
        _body,
        grid=grid,
        in_specs=[
            pl.BlockSpec((1, T, D), lambda bi, ci: (bi, ci, 0)),
            pl.BlockSpec((q, D), lambda bi, ci: (ci, 0)),
            pl.BlockSpec((c, D), lambda bi, ci: (0, 0)),
        ],
        out_specs=pl.BlockSpec((1, T, D), lambda bi, ci: (bi, ci, 0)),
        out_shape=jax.ShapeDtypeStruct((b, c * T, D), x.dtype),
    )(x.reshape(b, c * T, D), time_table, channel_table)
    return out


# speedup vs baseline: 3.9150x; 2.2158x over previous
import jax
import jax.numpy as jnp
from jax.experimental import pallas as pl


def _body(x_ref, tt_ref, ct_ref, o_ref):
    q, d = tt_ref.shape
    c = ct_ref.shape[0]
    xv = x_ref[0].reshape(q, c, d)
    tt = tt_ref[...]
    ct = ct_ref[...]
    o_ref[0] = (xv + tt[:, None, :] + ct[None, :, :]).reshape(q * c, d)


def kernel(x, time_table, channel_table):
    b, c, T, D = x.shape
    assert T % c == 0
    q = T // c
    CB = 8
    grid = (b, c // CB)
    out = pl.pallas_call(
        _body,
        grid=grid,
        in_specs=[
            pl.BlockSpec((1, CB * T, D), lambda bi, ci: (bi, ci, 0)),
            pl.BlockSpec((CB * q, D), lambda bi, ci: (ci, 0)),
            pl.BlockSpec((c, D), lambda bi, ci: (0, 0)),
        ],
        out_specs=pl.BlockSpec((1, CB * T, D), lambda bi, ci: (bi, ci, 0)),
        out_shape=jax.ShapeDtypeStruct((b, c * T, D), x.dtype),
    )(x.reshape(b, c * T, D), time_table, channel_table)
    return out


# speedup vs baseline: 3.9526x; 1.0096x over previous
import jax
import jax.numpy as jnp
from jax.experimental import pallas as pl


def _body(x_ref, tt_ref, ct_ref, o_ref):
    q, d = tt_ref.shape
    c = ct_ref.shape[0]
    xv = x_ref[0].reshape(q, c, d)
    tt = tt_ref[...]
    ct = ct_ref[...]
    o_ref[0] = (xv + tt[:, None, :] + ct[None, :, :]).reshape(q * c, d)


def kernel(x, time_table, channel_table):
    b, c, T, D = x.shape
    assert T % c == 0
    q = T // c
    CB = 16
    grid = (b, c // CB)
    out = pl.pallas_call(
        _body,
        grid=grid,
        in_specs=[
            pl.BlockSpec((1, CB * T, D), lambda bi, ci: (bi, ci, 0)),
            pl.BlockSpec((CB * q, D), lambda bi, ci: (ci, 0)),
            pl.BlockSpec((c, D), lambda bi, ci: (0, 0)),
        ],
        out_specs=pl.BlockSpec((1, CB * T, D), lambda bi, ci: (bi, ci, 0)),
        out_shape=jax.ShapeDtypeStruct((b, c * T, D), x.dtype),
    )(x.reshape(b, c * T, D), time_table, channel_table)
    return out
